# initial kernel scaffold (unmeasured)
import jax
import jax.numpy as jnp
from jax import lax
from jax.experimental import pallas as pl
from jax.experimental.pallas import tpu as pltpu

VOCAB_PER_SHARD = 8192


def kernel(ids, E):
    x = lax.axis_index("x")
    vocab_start = x * VOCAB_PER_SHARD
    local_idx = jnp.clip(ids - vocab_start, 0, VOCAB_PER_SHARD - 1)
    mine = (ids >= vocab_start) & (ids < vocab_start + VOCAB_PER_SHARD)
    partial = jnp.where(
        mine[:, None], jnp.take(E, local_idx, axis=0), jnp.float32(0.0)
    )

    def body(in_ref, out_ref, send_sem, recv_sem):
        bx = lax.axis_index("x")
        by = lax.axis_index("y")
        bz = lax.axis_index("z")
        partner = (1 - bx, by, bz)

        barrier = pltpu.get_barrier_semaphore()
        pl.semaphore_signal(
            barrier, inc=1, device_id=partner,
            device_id_type=pl.DeviceIdType.MESH,
        )
        pl.semaphore_wait(barrier, 1)

        rdma = pltpu.make_async_remote_copy(
            src_ref=in_ref,
            dst_ref=out_ref,
            send_sem=send_sem,
            recv_sem=recv_sem,
            device_id=partner,
            device_id_type=pl.DeviceIdType.MESH,
        )
        rdma.start()
        rdma.wait()

    other = pl.pallas_call(
        body,
        out_shape=jax.ShapeDtypeStruct(partial.shape, jnp.float32),
        in_specs=[pl.BlockSpec(memory_space=pltpu.ANY)],
        out_specs=pl.BlockSpec(memory_space=pltpu.ANY),
        scratch_shapes=[
            pltpu.SemaphoreType.DMA,
            pltpu.SemaphoreType.DMA,
        ],
        compiler_params=pltpu.CompilerParams(collective_id=0),
    )(partial)
    return partial + other


# baseline (device time: 2835001 ns/iter reference)
import jax
import jax.numpy as jnp
from jax import lax
from jax.experimental import pallas as pl
from jax.experimental.pallas import tpu as pltpu

VOCAB_PER_SHARD = 8192


def kernel(ids, E):
    x = lax.axis_index("x")
    vocab_start = x * VOCAB_PER_SHARD
    local_idx = jnp.clip(ids - vocab_start, 0, VOCAB_PER_SHARD - 1)
    mine = (ids >= vocab_start) & (ids < vocab_start + VOCAB_PER_SHARD)
    partial = jnp.where(
        mine[:, None], jnp.take(E, local_idx, axis=0), jnp.float32(0.0)
    )

    def body(in_ref, out_ref, send_sem, recv_sem):
        bx = lax.axis_index("x")
        by = lax.axis_index("y")
        bz = lax.axis_index("z")
        partner = (1 - bx, by, bz)

        barrier = pltpu.get_barrier_semaphore()
        pl.semaphore_signal(
            barrier, inc=1, device_id=partner,
            device_id_type=pl.DeviceIdType.MESH,
        )
        pl.semaphore_wait(barrier, 1)

        rdma = pltpu.make_async_remote_copy(
            src_ref=in_ref,
            dst_ref=out_ref,
            send_sem=send_sem,
            recv_sem=recv_sem,
            device_id=partner,
            device_id_type=pl.DeviceIdType.MESH,
        )
        rdma.start()
        rdma.wait()

    other = pl.pallas_call(
        body,
        out_shape=jax.ShapeDtypeStruct(partial.shape, jnp.float32),
        in_specs=[pl.BlockSpec(memory_space=pltpu.MemorySpace.HBM)],
        out_specs=pl.BlockSpec(memory_space=pltpu.MemorySpace.HBM),
        scratch_shapes=[
            pltpu.SemaphoreType.DMA,
            pltpu.SemaphoreType.DMA,
        ],
        compiler_params=pltpu.CompilerParams(collective_id=0),
    )(partial)
    return partial + other


# device time: 236164 ns/iter; 12.0044x vs baseline; 12.0044x over previous
import jax
import jax.numpy as jnp
from jax import lax
from jax.experimental import pallas as pl
from jax.experimental.pallas import tpu as pltpu

T = 4096
V_PER = 8192
D = 2048
CH = 128
MAXC = 20
RING = 64


def kernel(ids, E):
    iota = lax.iota(jnp.int32, T)
    sids, perm = lax.sort_key_val(ids.astype(jnp.int32), iota)
    c0 = jnp.sum((ids < V_PER).astype(jnp.int32)).reshape((1,))
    x = lax.axis_index("x")
    sids_local = jnp.clip(sids - x * V_PER, 0, V_PER - 1).astype(jnp.int32)

    def body(sids_ref, perm_ref, c0_ref, e_ref, out_ref, cme, cpr, ssem,
             rsem, gsem, ring_a, ring_b):
        bx = lax.axis_index("x")
        by = lax.axis_index("y")
        bz = lax.axis_index("z")
        partner = (1 - bx, by, bz)

        cnt0 = c0_ref[0]
        s_me = jnp.where(bx == 0, 0, cnt0)
        m_me = jnp.where(bx == 0, cnt0, T - cnt0)
        s_pr = jnp.where(bx == 0, cnt0, 0)
        m_pr = T - m_me
        n_me = (m_me + CH - 1) // CH
        n_pr = (m_pr + CH - 1) // CH

        def send_desc(c):
            return pltpu.make_async_remote_copy(
                src_ref=cme.at[pl.ds(c * CH, CH)],
                dst_ref=cpr.at[pl.ds(c * CH, CH)],
                send_sem=ssem.at[c],
                recv_sem=rsem.at[c],
                device_id=partner,
                device_id_type=pl.DeviceIdType.MESH,
            )

        barrier = pltpu.get_barrier_semaphore()
        pl.semaphore_signal(
            barrier, inc=1, device_id=partner,
            device_id_type=pl.DeviceIdType.MESH,
        )
        pl.semaphore_wait(barrier, 1)

        def gather_desc(c, j):
            k = c * CH + j
            src = sids_ref[jnp.minimum(s_me + k, s_me + m_me - 1)]
            return pltpu.make_async_copy(
                e_ref.at[pl.ds(src, 1)], cme.at[pl.ds(k, 1)], gsem.at[j]
            )

        for c in range(MAXC):
            @pl.when(c < n_me)
            def _(c=c):
                def issue(j, _):
                    gather_desc(c, j).start()
                    return 0

                lax.fori_loop(0, CH, issue, 0)

                def drain(j, _):
                    gather_desc(c, j).wait()
                    return 0

                lax.fori_loop(0, CH, drain, 0)
                send_desc(c).start()

        def scat_a_desc(k):
            return pltpu.make_async_copy(
                cme.at[pl.ds(k, 1)],
                out_ref.at[pl.ds(perm_ref[s_me + k], 1)],
                ring_a.at[k % RING],
            )

        def scat_a(k, _):
            @pl.when(k >= RING)
            def _():
                scat_a_desc(k - RING).wait()

            scat_a_desc(k).start()
            return 0

        lax.fori_loop(0, m_me, scat_a, 0)

        na = jnp.minimum(m_me, RING)

        def drain_a(j, _):
            scat_a_desc(m_me - na + j).wait()
            return 0

        lax.fori_loop(0, na, drain_a, 0)

        def scat_b_desc(k):
            return pltpu.make_async_copy(
                cpr.at[pl.ds(k, 1)],
                out_ref.at[pl.ds(perm_ref[s_pr + k], 1)],
                ring_b.at[k % RING],
            )

        def scat_b(k, _):
            @pl.when(k >= RING)
            def _():
                scat_b_desc(k - RING).wait()

            scat_b_desc(k).start()
            return 0

        for c in range(MAXC):
            @pl.when(c < n_pr)
            def _(c=c):
                send_desc(c).wait_recv()
                lax.fori_loop(
                    c * CH, jnp.minimum((c + 1) * CH, m_pr), scat_b, 0
                )

        nb = jnp.minimum(m_pr, RING)

        def drain_b(j, _):
            scat_b_desc(m_pr - nb + j).wait()
            return 0

        lax.fori_loop(0, nb, drain_b, 0)

        for c in range(MAXC):
            @pl.when(c < n_me)
            def _(c=c):
                send_desc(c).wait_send()

    return pl.pallas_call(
        body,
        out_shape=jax.ShapeDtypeStruct((T, D), jnp.float32),
        in_specs=[
            pl.BlockSpec(memory_space=pltpu.MemorySpace.SMEM),
            pl.BlockSpec(memory_space=pltpu.MemorySpace.SMEM),
            pl.BlockSpec(memory_space=pltpu.MemorySpace.SMEM),
            pl.BlockSpec(memory_space=pltpu.MemorySpace.HBM),
        ],
        out_specs=pl.BlockSpec(memory_space=pltpu.MemorySpace.HBM),
        scratch_shapes=[
            pltpu.VMEM((MAXC * CH, D), jnp.float32),
            pltpu.VMEM((MAXC * CH, D), jnp.float32),
            pltpu.SemaphoreType.DMA((MAXC,)),
            pltpu.SemaphoreType.DMA((MAXC,)),
            pltpu.SemaphoreType.DMA((CH,)),
            pltpu.SemaphoreType.DMA((RING,)),
            pltpu.SemaphoreType.DMA((RING,)),
        ],
        compiler_params=pltpu.CompilerParams(
            collective_id=0, vmem_limit_bytes=60 * 1024 * 1024
        ),
    )(sids_local, perm, c0, E)
